# baseline (speedup 1.0000x reference)
import functools

import jax
import jax.numpy as jnp
from jax import lax
from jax.experimental import pallas as pl
from jax.experimental.pallas import tpu as pltpu
from jax.experimental.pallas import tpu_sc as plsc

B = 4096
L = 200
H = 64
NW = 32
U_PER_W = B // NW
I_PER_W = U_PER_W * L
NGRP = 14
NPAIR = NGRP // 2
ROWS_PAD = NGRP * 16
OPAD = 24
SPLIT = 104
NSLOT = 2


def _sc_body(uf_hbm, item_hbm, mask_hbm, itf_hbm, out_hbm,
             uf_v, iidx_v, mask_v, rows_v, out_v, ufd, sem):
    wid = lax.axis_index("s") * 2 + lax.axis_index("c")
    ubase = wid * U_PER_W
    ibase = wid * I_PER_W

    pltpu.sync_copy(uf_hbm.at[pl.ds(ubase, U_PER_W)], uf_v)
    pltpu.sync_copy(item_hbm.at[pl.ds(ibase, I_PER_W)], iidx_v)
    pltpu.sync_copy(mask_hbm.at[pl.ds(ibase, I_PER_W)],
                    mask_v.at[pl.ds(0, I_PER_W)])

    iota = lax.iota(jnp.int32, 16)

    def issue(u, soff, ph):
        off = u * L
        pltpu.async_copy(
            itf_hbm.at[iidx_v.at[pl.ds(off, SPLIT)]],
            rows_v.at[pl.ds(soff, SPLIT)], sem.at[ph])
        pltpu.async_copy(
            itf_hbm.at[iidx_v.at[pl.ds(off + SPLIT, L - SPLIT)]],
            rows_v.at[pl.ds(soff + SPLIT, L - SPLIT)], sem.at[ph])

    def wait(soff, ph):
        pltpu.make_async_copy(
            itf_hbm.at[pl.ds(0, L)],
            rows_v.at[pl.ds(soff, L)], sem.at[ph]).wait()

    def compute(u, soff):
        for c in range(4):
            ufd[pl.ds(c * 16, 16)] = uf_v[u, pl.ds(c * 16, 16)]
        ufd[pl.ds(64, 16)] = uf_v[u, pl.ds(0, 16)]
        obase = u * L

        def pair_body(p, carry):
            ga = 2 * p
            idx0a = iota + (soff + ga * 16)
            idx0b = idx0a + 16
            acc = [jnp.zeros((16,), jnp.float32) for _ in range(8)]
            for h in range(H):
                rot = jnp.bitwise_and(iota + h, 63)
                m = ufd[pl.ds(h, 16)]
                cola = plsc.load_gather(rows_v, [idx0a, rot])
                colb = plsc.load_gather(rows_v, [idx0b, rot])
                acc[h % 4] = acc[h % 4] + m * cola
                acc[4 + h % 4] = acc[4 + h % 4] + m * colb
            tota = (acc[0] + acc[1]) + (acc[2] + acc[3])
            totb = (acc[4] + acc[5]) + (acc[6] + acc[7])
            oa = obase + ga * 16
            ma = mask_v[pl.ds(oa, 16)]
            mb = mask_v[pl.ds(oa + 16, 16)]
            out_v[pl.ds(oa, 16)] = jnp.where(
                ma == 0, jnp.float32(-1.0e13), tota)
            out_v[pl.ds(oa + 16, 16)] = jnp.where(
                mb == 0, jnp.float32(-1.0e13), totb)
            return carry

        lax.fori_loop(0, NPAIR, pair_body, 0)

    issue(0, 0, 0)
    issue(1, ROWS_PAD, 1)

    def u_body(u, carry):
        ph = lax.rem(u, NSLOT)
        soff = ph * ROWS_PAD
        wait(soff, ph)
        compute(u, soff)

        @pl.when(u < U_PER_W - NSLOT)
        def _():
            issue(u + NSLOT, soff, ph)
        return carry

    lax.fori_loop(0, U_PER_W, u_body, 0)

    pltpu.sync_copy(out_v.at[pl.ds(0, I_PER_W)],
                    out_hbm.at[pl.ds(ibase, I_PER_W)])


@functools.partial(
    pl.kernel,
    out_type=jax.ShapeDtypeStruct((B * L,), jnp.float32),
    mesh=plsc.VectorSubcoreMesh(core_axis_name="c", subcore_axis_name="s"),
    scratch_types=[
        pltpu.VMEM((U_PER_W, H), jnp.float32),
        pltpu.VMEM((I_PER_W,), jnp.int32),
        pltpu.VMEM((I_PER_W + OPAD,), jnp.int32),
        pltpu.VMEM((NSLOT * ROWS_PAD, H), jnp.float32),
        pltpu.VMEM((I_PER_W + OPAD,), jnp.float32),
        pltpu.VMEM((80,), jnp.float32),
        pltpu.SemaphoreType.DMA((NSLOT,)),
    ],
    compiler_params=pltpu.CompilerParams(needs_layout_passes=False,
                                         use_tc_tiling_on_sc=False),
)
def _sc_kernel(uf_hbm, item_hbm, mask_hbm, itf_hbm, out_hbm,
               uf_v, iidx_v, mask_v, rows_v, out_v, ufd, sem):
    _sc_body(uf_hbm, item_hbm, mask_hbm, itf_hbm, out_hbm,
             uf_v, iidx_v, mask_v, rows_v, out_v, ufd, sem)


def kernel(user, item, mask, user_factors, item_factors):
    uf_rows = jnp.take(user_factors, user, axis=0, mode="fill",
                       fill_value=0.0)
    logits = _sc_kernel(uf_rows,
                        item.reshape(-1).astype(jnp.int32),
                        mask.reshape(-1).astype(jnp.int32),
                        item_factors)
    return logits.reshape(B, L)


# speedup vs baseline: 1.0281x; 1.0281x over previous
import functools

import jax
import jax.numpy as jnp
from jax import lax
from jax.experimental import pallas as pl
from jax.experimental.pallas import tpu as pltpu
from jax.experimental.pallas import tpu_sc as plsc

B = 4096
L = 200
H = 64
NW = 32
U_PER_W = B // NW
I_PER_W = U_PER_W * L
NGRP = 13
ROWS_PAD = NGRP * 16
OPAD = 8
SPLIT = 104
NSLOT = 2


def _rot(iota, h):
    r = iota + h
    return r if h <= 48 else jnp.bitwise_and(r, 63)


def _sc_body(uf_hbm, item_hbm, mask_hbm, itf_hbm, out_hbm,
             uf_v, iidx_v, mask_v, rows_v, out_v, ufd, sem):
    wid = lax.axis_index("s") * 2 + lax.axis_index("c")
    ubase = wid * U_PER_W
    ibase = wid * I_PER_W

    iota = lax.iota(jnp.int32, 16)

    def issue(u, soff, ph):
        off = u * L
        pltpu.async_copy(
            itf_hbm.at[iidx_v.at[pl.ds(off, SPLIT)]],
            rows_v.at[pl.ds(soff, SPLIT)], sem.at[ph])
        pltpu.async_copy(
            itf_hbm.at[iidx_v.at[pl.ds(off + SPLIT, L - SPLIT)]],
            rows_v.at[pl.ds(soff + SPLIT, L - SPLIT)], sem.at[ph])

    def wait(soff, ph):
        pltpu.make_async_copy(
            itf_hbm.at[pl.ds(0, L)],
            rows_v.at[pl.ds(soff, L)], sem.at[ph]).wait()

    pltpu.sync_copy(item_hbm.at[pl.ds(ibase, I_PER_W)], iidx_v)
    issue(0, 0, 0)
    issue(1, ROWS_PAD, 1)
    pltpu.sync_copy(uf_hbm.at[pl.ds(ubase, U_PER_W)], uf_v)
    pltpu.sync_copy(mask_hbm.at[pl.ds(ibase, I_PER_W)],
                    mask_v.at[pl.ds(0, I_PER_W)])

    def group_epilogue(obase, g, tot):
        o = obase + g * 16
        m = mask_v[pl.ds(o, 16)]
        out_v[pl.ds(o, 16)] = jnp.where(m == 0, jnp.float32(-1.0e13), tot)

    def compute(u, soff):
        for c in range(4):
            ufd[pl.ds(c * 16, 16)] = uf_v[u, pl.ds(c * 16, 16)]
        ufd[pl.ds(64, 16)] = uf_v[u, pl.ds(0, 16)]
        obase = u * L

        def quad_body(q, carry):
            base = soff + q * 64
            idx0 = [iota + (base + k * 16) for k in range(4)]
            acc = [jnp.zeros((16,), jnp.float32) for _ in range(16)]
            for h in range(H):
                rot = _rot(iota, h)
                m = ufd[pl.ds(h, 16)]
                for k in range(4):
                    col = plsc.load_gather(rows_v, [idx0[k], rot])
                    a = 4 * k + h % 4
                    acc[a] = acc[a] + m * col
            for k in range(4):
                tot = ((acc[4 * k] + acc[4 * k + 1])
                       + (acc[4 * k + 2] + acc[4 * k + 3]))
                group_epilogue(obase, 4 * q + k, tot)
            return carry

        lax.fori_loop(0, 3, quad_body, 0)

        idx0t = iota + (soff + 192)
        acc = [jnp.zeros((16,), jnp.float32) for _ in range(4)]
        for h in range(H):
            col = plsc.load_gather(rows_v, [idx0t, _rot(iota, h)])
            acc[h % 4] = acc[h % 4] + ufd[pl.ds(h, 16)] * col
        group_epilogue(obase, 12, (acc[0] + acc[1]) + (acc[2] + acc[3]))

    def u_body(u, carry):
        ph = lax.rem(u, NSLOT)
        soff = ph * ROWS_PAD
        wait(soff, ph)
        compute(u, soff)

        @pl.when(u < U_PER_W - NSLOT)
        def _():
            issue(u + NSLOT, soff, ph)
        return carry

    lax.fori_loop(0, U_PER_W, u_body, 0)

    pltpu.sync_copy(out_v.at[pl.ds(0, I_PER_W)],
                    out_hbm.at[pl.ds(ibase, I_PER_W)])


@functools.partial(
    pl.kernel,
    out_type=jax.ShapeDtypeStruct((B * L,), jnp.float32),
    mesh=plsc.VectorSubcoreMesh(core_axis_name="c", subcore_axis_name="s"),
    scratch_types=[
        pltpu.VMEM((U_PER_W, H), jnp.float32),
        pltpu.VMEM((I_PER_W,), jnp.int32),
        pltpu.VMEM((I_PER_W + OPAD,), jnp.int32),
        pltpu.VMEM((NSLOT * ROWS_PAD, H), jnp.float32),
        pltpu.VMEM((I_PER_W + OPAD,), jnp.float32),
        pltpu.VMEM((80,), jnp.float32),
        pltpu.SemaphoreType.DMA((NSLOT,)),
    ],
    compiler_params=pltpu.CompilerParams(needs_layout_passes=False,
                                         use_tc_tiling_on_sc=False),
)
def _sc_kernel(uf_hbm, item_hbm, mask_hbm, itf_hbm, out_hbm,
               uf_v, iidx_v, mask_v, rows_v, out_v, ufd, sem):
    _sc_body(uf_hbm, item_hbm, mask_hbm, itf_hbm, out_hbm,
             uf_v, iidx_v, mask_v, rows_v, out_v, ufd, sem)


def kernel(user, item, mask, user_factors, item_factors):
    uf_rows = jnp.take(user_factors, user, axis=0)
    logits = _sc_kernel(uf_rows,
                        item.reshape(-1).astype(jnp.int32),
                        mask.reshape(-1).astype(jnp.int32),
                        item_factors)
    return logits.reshape(B, L)
